# baseline (speedup 1.0000x reference)
import functools

import numpy as np
import jax
import jax.numpy as jnp
from jax import lax
from jax.experimental import pallas as pl
from jax.experimental.pallas import tpu as pltpu
from jax.experimental.pallas import tpu_sc as plsc

_BINS = 10
_MOM = np.float32(0.75)
_NC, _NS = 2, 16
_NW = _NC * _NS


def _sc_gather(flat, idx3):
    nw, ch, lanes = idx3.shape
    mesh = plsc.VectorSubcoreMesh(core_axis_name="c", subcore_axis_name="s",
                                  num_cores=_NC, num_subcores=_NS)

    @functools.partial(
        pl.kernel, mesh=mesh,
        out_type=jax.ShapeDtypeStruct((nw, ch, lanes), jnp.float32),
        scratch_types=[
            pltpu.VMEM((ch, lanes), jnp.int32),
            pltpu.VMEM((ch, lanes), jnp.float32),
            pltpu.SemaphoreType.DMA,
        ],
    )
    def _g(flat_hbm, idx_hbm, out_hbm, idx_v, val_v, sem):
        wid = lax.axis_index("s") * _NC + lax.axis_index("c")
        pltpu.sync_copy(idx_hbm.at[wid], idx_v)
        cps = [pltpu.async_copy(flat_hbm.at[idx_v.at[j]], val_v.at[j], sem)
               for j in range(ch)]
        for cp in cps:
            cp.wait()
        pltpu.sync_copy(val_v, out_hbm.at[wid])

    return _g(flat, idx3)


def _stats_body(logits_ref, s_ref, *, rows):
    x = logits_ref[...]
    m = jnp.max(x, axis=1)
    e = jnp.exp(x - m[:, None])
    z = jnp.sum(e, axis=1)
    s_ref[...] = (m + jnp.log(z)).reshape(1, 1, rows)


def _finish_body(s_ref, xl_ref, acc_ref, out_ref, *, total, n_elems):
    s2 = s_ref[...]
    xl2 = xl_ref[...]
    u = xl2 - s2
    ce = -u
    g = np.float32(1) - jnp.exp(u)
    binv = jnp.zeros(g.shape, jnp.int32)
    for j in range(_BINS):
        binv = binv + (g > np.float32(j) / np.float32(10)).astype(jnp.int32)
    w = jnp.zeros(g.shape, jnp.float32)
    for k in range(_BINS):
        mk = binv == k
        c_k = jnp.sum(mk.astype(jnp.float32))
        a_k = acc_ref[k]
        a_new = jnp.where(c_k > 0, _MOM * a_k + (np.float32(1) - _MOM) * c_k, a_k)
        w_k = jnp.where(c_k > 0, total / a_new, np.float32(0))
        w = w + jnp.where(mk, w_k, np.float32(0))
    wsum = jnp.sum(w)
    loss = jnp.sum(ce * w)
    out_ref[...] = jnp.reshape(loss / wsum * (total / n_elems), (1, 1))


def kernel(logits, labels, acc_sum):
    n, c = logits.shape
    rows = 512
    nblk = n // rows
    bpw = n // _NW
    ch = bpw // 128

    flat = logits.reshape(-1)
    idx3 = (jnp.arange(n, dtype=jnp.int32) * c + labels).reshape(_NW, ch, 128)
    xl = _sc_gather(flat, idx3)

    s = pl.pallas_call(
        functools.partial(_stats_body, rows=rows),
        grid=(nblk,),
        in_specs=[pl.BlockSpec((rows, c), lambda i: (i, 0))],
        out_specs=pl.BlockSpec((1, 1, rows), lambda i: (i, 0, 0)),
        out_shape=jax.ShapeDtypeStruct((nblk, 1, rows), jnp.float32),
        compiler_params=pltpu.CompilerParams(dimension_semantics=("arbitrary",)),
    )(logits)

    total = np.float32(n)
    side = 128
    out = pl.pallas_call(
        functools.partial(_finish_body, total=total, n_elems=np.float32(n)),
        in_specs=[
            pl.BlockSpec((side, n // side), lambda: (0, 0)),
            pl.BlockSpec((side, n // side), lambda: (0, 0)),
            pl.BlockSpec(memory_space=pltpu.SMEM),
        ],
        out_specs=pl.BlockSpec((1, 1), lambda: (0, 0)),
        out_shape=jax.ShapeDtypeStruct((1, 1), jnp.float32),
    )(s.reshape(side, n // side), xl.reshape(side, n // side), acc_sum)
    return out[0, 0]


# speedup vs baseline: 1.9753x; 1.9753x over previous
import functools

import numpy as np
import jax
import jax.numpy as jnp
from jax import lax
from jax.experimental import pallas as pl
from jax.experimental.pallas import tpu as pltpu

_BINS = 10
_MOM = np.float32(0.75)


def _ghm_body(logits_ref, labels_ref, acc_ref, out_ref, g_scr, ce_scr,
              *, nblk, rows, ncls, total):
    i = pl.program_id(0)
    x = logits_ref[...]
    lab = labels_ref[0, 0, :]
    m = jnp.max(x, axis=1)
    e = jnp.exp(x - m[:, None])
    z = jnp.sum(e, axis=1)
    col = lax.broadcasted_iota(jnp.int32, (rows, ncls), 1)
    sel = col == lab[:, None]
    xl = jnp.sum(jnp.where(sel, x, np.float32(0)), axis=1)
    u = xl - m
    ce = jnp.log(z) - u
    g = np.float32(1) - jnp.exp(u) / z
    g_scr[pl.ds(i, 1), :] = g.reshape(1, rows)
    ce_scr[pl.ds(i, 1), :] = ce.reshape(1, rows)

    @pl.when(i == nblk - 1)
    def _finish():
        gg = g_scr[...]
        cc = ce_scr[...]
        binv = jnp.zeros(gg.shape, jnp.int32)
        for j in range(_BINS):
            binv = binv + (gg > np.float32(j) / np.float32(10)).astype(jnp.int32)
        w = jnp.zeros(gg.shape, jnp.float32)
        for k in range(_BINS):
            mk = binv == k
            c_k = jnp.sum(mk.astype(jnp.float32))
            a_k = acc_ref[k]
            a_new = jnp.where(c_k > 0, _MOM * a_k + (np.float32(1) - _MOM) * c_k, a_k)
            w_k = jnp.where(c_k > 0, total / a_new, np.float32(0))
            w = w + jnp.where(mk, w_k, np.float32(0))
        wsum = jnp.sum(w)
        loss = jnp.sum(cc * w)
        n_elems = np.float32(nblk * rows)
        out_ref[...] = jnp.reshape(loss / wsum * (total / n_elems), (1, 1))


def kernel(logits, labels, acc_sum):
    n, c = logits.shape
    rows = 2048
    nblk = n // rows
    labels3 = labels.reshape(nblk, 1, rows)
    total = np.float32(n)
    body = functools.partial(_ghm_body, nblk=nblk, rows=rows, ncls=c, total=total)
    out = pl.pallas_call(
        body,
        grid=(nblk,),
        in_specs=[
            pl.BlockSpec((rows, c), lambda i: (i, 0)),
            pl.BlockSpec((1, 1, rows), lambda i: (i, 0, 0)),
            pl.BlockSpec(memory_space=pltpu.SMEM),
        ],
        out_specs=pl.BlockSpec((1, 1), lambda i: (0, 0)),
        out_shape=jax.ShapeDtypeStruct((1, 1), jnp.float32),
        scratch_shapes=[
            pltpu.VMEM((nblk, rows), jnp.float32),
            pltpu.VMEM((nblk, rows), jnp.float32),
        ],
        compiler_params=pltpu.CompilerParams(dimension_semantics=("arbitrary",)),
    )(logits, labels3, acc_sum)
    return out[0, 0]
